# Initial kernel scaffold; baseline (speedup 1.0000x reference)
import jax
import jax.numpy as jnp
from jax.experimental import pallas as pl


def kernel(x, edge_index, edge_attr, batch, W1, We1, as1, ad1, ae1, b1, W2, We2, as2, ad2, ae2, b2, W3, We3, as3, ad3, ae3, b3, W4, We4, as4, ad4, ae4, b4, Wl, bl):
    raise NotImplementedError("write your pallas kernel here")


# speedup vs baseline: 10.6157x; 10.6157x over previous
import functools

import jax
import jax.numpy as jnp
from jax import lax
from jax.experimental import pallas as pl
from jax.experimental.pallas import tpu as pltpu
from jax.experimental.pallas import tpu_sc as plsc

H = 8
HCDIM = 128
HHC = 1024
DE = 16
NGRP = 64
NCORE = 2
NSUB = 16
NWK = NCORE * NSUB
LANES = 16

F32 = jnp.float32
I32 = jnp.int32


def _elu(v):
    return jnp.where(v > 0, v, jnp.exp(v) - 1.0)


def _proj_call(u, wcat, bvec, apply_act, n_pad):
    bn = 512
    k = u.shape[1]

    def body(u_ref, w_ref, b_ref, *outs):
        uu = u_ref[...]
        if apply_act:
            uu = _elu(uu + b_ref[...])
        hs = lax.dot_general(uu, w_ref[...], (((1,), (0,)), ((), ())),
                             preferred_element_type=F32)
        for i in range(H):
            outs[i][...] = hs[:, HCDIM * i:HCDIM * (i + 1)]
        st = hs[:, HHC:HHC + 16]
        outs[H][...] = jnp.concatenate(
            [st, jnp.zeros((st.shape[0], HCDIM - 16), F32)], axis=1)
        outs[H + 1][...] = st

    grid = n_pad // bn
    out_shapes = [jax.ShapeDtypeStruct((n_pad, HCDIM), F32) for _ in range(H)]
    out_shapes.append(jax.ShapeDtypeStruct((n_pad, HCDIM), F32))
    out_shapes.append(jax.ShapeDtypeStruct((n_pad, 16), F32))
    out_specs = [pl.BlockSpec((bn, HCDIM), lambda j: (j, 0))
                 for _ in range(H + 1)]
    out_specs.append(pl.BlockSpec((bn, 16), lambda j: (j, 0)))
    return pl.pallas_call(
        body,
        grid=(grid,),
        in_specs=[
            pl.BlockSpec((bn, k), lambda j: (j, 0)),
            pl.BlockSpec((k, 1152), lambda j: (0, 0)),
            pl.BlockSpec((1, HHC), lambda j: (0, 0)),
        ],
        out_specs=out_specs,
        out_shape=out_shapes,
    )(u, wcat, bvec)


def _ealpha_call(ea, ae_all):
    e = ea.shape[0]
    be = 3200

    def body(ea_ref, ae_ref, *outs):
        q = lax.dot_general(ea_ref[...], ae_ref[...], (((1,), (0,)), ((), ())),
                            preferred_element_type=F32)
        for i in range(4):
            outs[i][...] = q[:, 8 * i:8 * (i + 1)]

    return pl.pallas_call(
        body,
        grid=(e // be,),
        in_specs=[
            pl.BlockSpec((be, DE), lambda j: (j, 0)),
            pl.BlockSpec((DE, 32), lambda j: (0, 0)),
        ],
        out_specs=[pl.BlockSpec((be, 8), lambda j: (j, 0)) for _ in range(4)],
        out_shape=[jax.ShapeDtypeStruct((e, 8), F32) for _ in range(4)],
    )(ea, ae_all)


def _qloop_call(p0, p1, ae_all, n_pad):
    bn = 1280

    def body(p0_ref, p1_ref, ae_ref, *outs):
        u = p0_ref[...] + p1_ref[...]
        deg = jnp.maximum(u[:, 16:17], 1.0)
        s16 = u[:, :16] / deg
        q = lax.dot_general(s16, ae_ref[...], (((1,), (0,)), ((), ())),
                            preferred_element_type=F32)
        for i in range(4):
            outs[i][...] = q[:, 8 * i:8 * (i + 1)]

    return pl.pallas_call(
        body,
        grid=(n_pad // bn,),
        in_specs=[
            pl.BlockSpec((bn, 32), lambda j: (j, 0)),
            pl.BlockSpec((bn, 32), lambda j: (j, 0)),
            pl.BlockSpec((DE, 32), lambda j: (0, 0)),
        ],
        out_specs=[pl.BlockSpec((bn, 8), lambda j: (j, 0)) for _ in range(4)],
        out_shape=[jax.ShapeDtypeStruct((n_pad, 8), F32) for _ in range(4)],
    )(p0, p1, ae_all)


def _mesh():
    return plsc.VectorSubcoreMesh(core_axis_name="c", subcore_axis_name="s")


def _wid(cc, ss):
    return ss * NCORE + cc


IOTA = lambda: lax.iota(I32, LANES)


def _make_pass0(n, n_pad, e):
    c0 = 400
    per_w = e // NWK
    nch = per_w // c0
    ndp = n_pad // 4
    rows_per_sub = ndp // NSUB

    @functools.partial(
        pl.kernel, mesh=_mesh(),
        compiler_params=pltpu.CompilerParams(needs_layout_passes=False),
        out_type=jax.ShapeDtypeStruct((2, ndp, 128), F32),
        scratch_types=[
            pltpu.VMEM_SHARED((ndp, 128), F32),
            pltpu.VMEM((c0 + 16,), I32),
            pltpu.VMEM((c0,), I32),
            pltpu.VMEM((c0, 128), F32),
            pltpu.VMEM((c0 * DE,), F32),
        ],
    )
    def k(ea_hbm, dst_hbm, out_hbm, acc_sp, didx, d4, pay_v, ea_v):
        cc = lax.axis_index("c")
        ss = lax.axis_index("s")
        wid = _wid(cc, ss)
        zv16 = jnp.zeros((16,), F32)
        onev = jnp.where(IOTA() == 0, 1.0, 0.0).astype(F32)

        def zp(i, _):
            for j in range(8):
                pay_v[i, pl.ds(j * 16, 16)] = zv16
            return 0
        lax.fori_loop(0, c0, zp, 0)

        for r0 in range(0, rows_per_sub, c0):
            rr = min(c0, rows_per_sub - r0)
            pltpu.sync_copy(
                pay_v.at[pl.ds(0, rr)],
                acc_sp.at[pl.ds(pl.multiple_of(ss * rows_per_sub + r0, 8),
                                rr)])
        plsc.subcore_barrier()

        def chunk(ci, _):
            base = wid * per_w + ci * c0
            pltpu.sync_copy(dst_hbm.at[pl.ds(base, c0)],
                            didx.at[pl.ds(0, c0)])
            pltpu.sync_copy(ea_hbm.at[pl.ds(base * DE, c0 * DE)], ea_v)

            def grp(g, _):
                dv = didx[pl.ds(g * LANES, LANES)]
                d4[pl.ds(g * LANES, LANES)] = lax.shift_right_logical(dv, 2)
                return 0
            lax.fori_loop(0, c0 // LANES, grp, 0)

            def ed(i, _):
                de = didx[pl.ds(i, 16)][0]
                col = (de & 3) * 32
                pay_v[i, pl.ds(col, 16)] = ea_v[pl.ds(i * DE, 16)]
                pay_v[i, pl.ds(col + 16, 16)] = onev
                return 0
            lax.fori_loop(0, c0, ed, 0)

            pltpu.sync_copy(pay_v, acc_sp.at[d4], add=True)

            def ed2(i, _):
                de = didx[pl.ds(i, 16)][0]
                col = (de & 3) * 32
                pay_v[i, pl.ds(col, 16)] = zv16
                pay_v[i, pl.ds(col + 16, 16)] = zv16
                return 0
            lax.fori_loop(0, c0, ed2, 0)
            return 0
        lax.fori_loop(0, nch, chunk, 0)

        plsc.subcore_barrier()
        row = pl.multiple_of(ss * rows_per_sub, 8)
        pltpu.sync_copy(acc_sp.at[pl.ds(row, rows_per_sub)],
                        out_hbm.at[cc, pl.ds(row, rows_per_sub)])

    return k


def _make_pass1(n, n_pad, e, e2p):
    c1 = 80
    per_w = e // NWK
    nch = per_w // c1
    ngrp = c1 // LANES
    ndp = n_pad // 8 + 128
    den_rows = ndp // NSUB
    pslf = n_pad // NWK

    @functools.partial(
        pl.kernel, mesh=_mesh(),
        compiler_params=pltpu.CompilerParams(needs_layout_passes=False),
        out_type=(
            [jax.ShapeDtypeStruct((e2p,), F32) for _ in range(H)]
            + [jax.ShapeDtypeStruct((2, ndp, 128), F32)]
        ),
        scratch_types=[
            pltpu.VMEM_SHARED((ndp, 128), F32),
            pltpu.VMEM((c1,), I32),
            pltpu.VMEM((c1,), I32),
            pltpu.VMEM((c1,), I32),
            pltpu.VMEM((c1, 128), F32),
            pltpu.VMEM((c1, 128), F32),
            pltpu.VMEM((c1 * 8,), F32),
            pltpu.VMEM((H, c1), F32),
            pltpu.VMEM((c1, 128), F32),
            pltpu.VMEM((den_rows, 128), F32),
            pltpu.VMEM((pslf * 16,), F32),
            pltpu.VMEM((pslf * 8,), F32),
        ],
    )
    def k(st128_hbm, st16_hbm, q_hbm, ql_hbm, src_hbm, dst_hbm,
          w0, w1, w2, w3, w4, w5, w6, w7, den_hbm,
          den_sp, sidx, didx, d8, sbuf, tbuf, qbuf, wbuf, pay_v, z_v,
          st16buf, qlbuf):
        cc = lax.axis_index("c")
        ss = lax.axis_index("s")
        wid = _wid(cc, ss)
        w_hbms = [w0, w1, w2, w3, w4, w5, w6, w7]
        zv16 = jnp.zeros((16,), F32)

        def zz(i, _):
            for j in range(8):
                z_v[i, pl.ds(j * 16, 16)] = zv16
            return 0
        lax.fori_loop(0, den_rows, zz, 0)

        def zp(i, _):
            for j in range(8):
                pay_v[i, pl.ds(j * 16, 16)] = zv16
            return 0
        lax.fori_loop(0, c1, zp, 0)

        pltpu.sync_copy(
            z_v, den_sp.at[pl.ds(pl.multiple_of(ss * den_rows, 8),
                                 den_rows)])
        plsc.subcore_barrier()

        def chunk(ci, _):
            base = wid * per_w + ci * c1
            pltpu.sync_copy(src_hbm.at[pl.ds(base, c1)], sidx)
            pltpu.sync_copy(dst_hbm.at[pl.ds(base, c1)], didx)
            pltpu.sync_copy(q_hbm.at[pl.ds(base * 8, c1 * 8)], qbuf)
            pltpu.sync_copy(st128_hbm.at[sidx], sbuf)
            pltpu.sync_copy(st128_hbm.at[didx], tbuf)

            def grp(g, _):
                rows = g * LANES + IOTA()
                dv = didx[pl.ds(g * LANES, LANES)]
                d8[pl.ds(g * LANES, LANES)] = lax.shift_right_logical(dv, 3)
                colv = (dv & 7) * 8
                for hh in range(H):
                    colh = jnp.full((LANES,), hh, I32)
                    sv = plsc.load_gather(sbuf, [rows, colh])
                    tv = plsc.load_gather(tbuf, [rows, colh + 8])
                    qv = plsc.load_gather(qbuf, [rows * 8 + hh])
                    al = sv + tv + qv
                    al = jnp.maximum(al, 0.2 * al)
                    wv = jnp.exp(al)
                    wbuf[hh, pl.ds(g * LANES, LANES)] = wv
                    plsc.store_scatter(pay_v, [rows, colv + hh], wv)
                return 0
            lax.fori_loop(0, ngrp, grp, 0)

            for hh in range(H):
                pltpu.sync_copy(wbuf.at[hh], w_hbms[hh].at[pl.ds(base, c1)])
            pltpu.sync_copy(pay_v, den_sp.at[d8], add=True)

            def clr(g, _):
                rows = g * LANES + IOTA()
                dv = didx[pl.ds(g * LANES, LANES)]
                colv = (dv & 7) * 8
                for hh in range(H):
                    plsc.store_scatter(pay_v, [rows, colv + hh],
                                       jnp.zeros((LANES,), F32))
                return 0
            lax.fori_loop(0, ngrp, clr, 0)
            return 0
        lax.fori_loop(0, nch, chunk, 0)

        nbase = wid * pslf
        pltpu.sync_copy(st16_hbm.at[pl.ds(nbase * 16, pslf * 16)], st16buf)
        pltpu.sync_copy(ql_hbm.at[pl.ds(nbase * 8, pslf * 8)], qlbuf)

        for sb in range(pslf // c1):
            def sgrp(g, _):
                rows = sb * c1 + g * LANES + IOTA()
                node = nbase + rows
                valid = node < n
                prow = g * LANES + IOTA()
                d8[pl.ds(g * LANES, LANES)] = lax.shift_right_logical(node, 3)
                colv = (node & 7) * 8
                for hh in range(H):
                    sv = plsc.load_gather(st16buf, [rows * 16 + hh])
                    tv = plsc.load_gather(st16buf, [rows * 16 + 8 + hh])
                    qv = plsc.load_gather(qlbuf, [rows * 8 + hh])
                    al = sv + tv + qv
                    al = jnp.maximum(al, 0.2 * al)
                    wv = jnp.where(valid, jnp.exp(al), 0.0)
                    wbuf[hh, pl.ds(g * LANES, LANES)] = wv
                    plsc.store_scatter(pay_v, [prow, colv + hh], wv)
                return 0
            lax.fori_loop(0, ngrp, sgrp, 0)

            for hh in range(H):
                pltpu.sync_copy(
                    wbuf.at[hh],
                    w_hbms[hh].at[pl.ds(e + nbase + sb * c1, c1)])
            pltpu.sync_copy(pay_v, den_sp.at[d8], add=True)

            def sclr(g, _):
                rows = sb * c1 + g * LANES + IOTA()
                node = nbase + rows
                prow = g * LANES + IOTA()
                colv = (node & 7) * 8
                for hh in range(H):
                    plsc.store_scatter(pay_v, [prow, colv + hh],
                                       jnp.zeros((LANES,), F32))
                return 0
            lax.fori_loop(0, ngrp, sclr, 0)

        plsc.subcore_barrier()
        drow = pl.multiple_of(ss * den_rows, 8)
        pltpu.sync_copy(den_sp.at[pl.ds(drow, den_rows)],
                        den_hbm.at[cc, pl.ds(drow, den_rows)])

    return k


def _make_pass2(n_pad, e2p):
    c2 = 120
    per_t = e2p // NSUB
    nch = per_t // c2
    na = n_pad + 16
    rb = 64
    rows_per_sub = n_pad // NSUB

    @functools.partial(
        pl.kernel, mesh=_mesh(),
        compiler_params=pltpu.CompilerParams(needs_layout_passes=False),
        out_type=jax.ShapeDtypeStruct((n_pad, HHC), F32),
        scratch_types=[
            pltpu.VMEM_SHARED((na, HCDIM), F32),
            pltpu.VMEM((c2,), I32),
            pltpu.VMEM((c2,), I32),
            pltpu.VMEM((c2 + 16, ), F32),
            pltpu.VMEM((c2, HCDIM), F32),
            pltpu.VMEM((rb, HCDIM), F32),
            pltpu.VMEM((rb // 8, 128), F32),
            pltpu.VMEM((rb // 8, 128), F32),
        ],
    )
    def k(h0, h1, h2, h3, h4, h5, h6, h7, src_hbm, dst_hbm,
          w0, w1, w2, w3, w4, w5, w6, w7, den_hbm,
          out_hbm, acc_sp, sidx, didx, wv_v, rows_v, eb_v, d0_v, d1_v):
        cc = lax.axis_index("c")
        ss = lax.axis_index("s")
        wid = _wid(cc, ss)
        htabs = [h0, h1, h2, h3, h4, h5, h6, h7]
        w_hbms = [w0, w1, w2, w3, w4, w5, w6, w7]
        zv16 = jnp.zeros((16,), F32)

        for head in range(H):
            sc = head // 4

            @pl.when(cc == sc)
            def _():
                def zz(i, _):
                    for j in range(HCDIM // LANES):
                        eb_v[i, pl.ds(j * LANES, LANES)] = zv16
                    return 0
                lax.fori_loop(0, rb, zz, 0)
                for r0 in range(0, rows_per_sub, rb):
                    pltpu.sync_copy(
                        eb_v,
                        acc_sp.at[pl.ds(
                            pl.multiple_of(ss * rows_per_sub + r0, 8), rb)])

                @pl.when(ss == 0)
                def _():
                    pltpu.sync_copy(eb_v.at[pl.ds(0, 16)],
                                    acc_sp.at[pl.ds(n_pad, 16)])
                plsc.subcore_barrier()

                def chunk(ci, _):
                    base = ss * per_t + ci * c2
                    pltpu.sync_copy(src_hbm.at[pl.ds(base, c2)], sidx)
                    pltpu.sync_copy(dst_hbm.at[pl.ds(base, c2)], didx)
                    pltpu.sync_copy(w_hbms[head].at[pl.ds(base, c2)],
                                    wv_v.at[pl.ds(0, c2)])
                    pltpu.sync_copy(htabs[head].at[sidx], rows_v)

                    def edge(ei, _):
                        we = wv_v[pl.ds(ei, 16)][0]
                        for j in range(HCDIM // LANES):
                            v = rows_v[ei, pl.ds(j * LANES, LANES)]
                            rows_v[ei, pl.ds(j * LANES, LANES)] = v * we
                        return 0
                    lax.fori_loop(0, c2, edge, 0)

                    pltpu.sync_copy(rows_v, acc_sp.at[didx], add=True)
                    return 0
                lax.fori_loop(0, nch, chunk, 0)
                plsc.subcore_barrier()

                for r0 in range(0, rows_per_sub, rb):
                    row0 = pl.multiple_of(ss * rows_per_sub + r0, 8)
                    row8 = pl.multiple_of((ss * rows_per_sub + r0) // 8, 8)
                    pltpu.sync_copy(acc_sp.at[pl.ds(row0, rb)], eb_v)
                    pltpu.sync_copy(den_hbm.at[0, pl.ds(row8, rb // 8)],
                                    d0_v)
                    pltpu.sync_copy(den_hbm.at[1, pl.ds(row8, rb // 8)],
                                    d1_v)

                    def erow(i, _):
                        pr = lax.shift_right_logical(i, 3)
                        pc = (i & 7) * 8
                        d0r = d0_v[pr, pl.ds(pc, 16)]
                        d1r = d1_v[pr, pl.ds(pc, 16)]
                        invv = 1.0 / (d0r + d1r + 1e-16)
                        inv = invv[head]
                        for j in range(HCDIM // LANES):
                            v = eb_v[i, pl.ds(j * LANES, LANES)]
                            eb_v[i, pl.ds(j * LANES, LANES)] = v * inv
                        return 0
                    lax.fori_loop(0, rb, erow, 0)

                    pltpu.sync_copy(
                        eb_v,
                        out_hbm.at[pl.ds(row0, rb),
                                   pl.ds(head * HCDIM, HCDIM)])
                plsc.subcore_barrier()

    return k


def _make_pass3(n, n_pad):
    pslf = n_pad // NWK
    sbn = 80
    cn = 8
    prows = 128
    prow_per_sub = prows // NSUB

    @functools.partial(
        pl.kernel, mesh=_mesh(),
        compiler_params=pltpu.CompilerParams(needs_layout_passes=False),
        out_type=jax.ShapeDtypeStruct((2, prows, 128), F32),
        scratch_types=[
            pltpu.VMEM_SHARED((prows, 128), F32),
            pltpu.VMEM((cn, HHC), F32),
            pltpu.VMEM((sbn, 128), F32),
            pltpu.VMEM((sbn,), I32),
            pltpu.VMEM((HCDIM,), F32),
            pltpu.VMEM((HCDIM,), F32),
        ],
    )
    def k(agg_hbm, b_hbm, wl_hbm, batch_hbm, out_hbm,
          pool_sp, ab_v, pay_v, bidx, b4_v, wl_v):
        cc = lax.axis_index("c")
        ss = lax.axis_index("s")
        wid = _wid(cc, ss)
        nbase = wid * pslf
        zv16 = jnp.zeros((16,), F32)

        def zp(i, _):
            for j in range(8):
                pay_v[i, pl.ds(j * 16, 16)] = zv16
            return 0
        lax.fori_loop(0, sbn, zp, 0)

        pltpu.sync_copy(
            pay_v.at[pl.ds(0, prow_per_sub)],
            pool_sp.at[pl.ds(pl.multiple_of(ss * prow_per_sub, 8),
                             prow_per_sub)])
        pltpu.sync_copy(b_hbm, b4_v)
        pltpu.sync_copy(wl_hbm, wl_v)
        plsc.subcore_barrier()

        for sb in range(pslf // sbn):
            sb_base = nbase + sb * sbn
            pltpu.sync_copy(batch_hbm.at[pl.ds(sb_base, sbn)], bidx)

            def chunk(ci, _):
                pltpu.sync_copy(agg_hbm.at[pl.ds(sb_base + ci * cn, cn)],
                                ab_v)

                def node(i, _):
                    zacc = 0.0
                    for j in range(HCDIM // LANES):
                        acc = ab_v[i, pl.ds(j * LANES, LANES)]
                        for hh in range(1, H):
                            acc = acc + ab_v[i, pl.ds(hh * HCDIM + j * LANES,
                                                      LANES)]
                        o = acc * (1.0 / H) + b4_v[pl.ds(j * LANES, LANES)]
                        o = jnp.where(o > 0, o, jnp.exp(o) - 1.0)
                        zacc = zacc + jnp.sum(o * wl_v[pl.ds(j * LANES,
                                                             LANES)])
                    node_id = sb_base + ci * cn + i
                    validf = jnp.where(node_id < n, 1.0, 0.0)
                    lane = IOTA()
                    row = (jnp.where(lane == 0, zacc * validf, 0.0)
                           + jnp.where(lane == 1, validf, 0.0))
                    pay_v[ci * cn + i, pl.ds(0, 16)] = row.astype(F32)
                    return 0
                lax.fori_loop(0, cn, node, 0)
                return 0
            lax.fori_loop(0, sbn // cn, chunk, 0)

            pltpu.sync_copy(pay_v, pool_sp.at[bidx], add=True)
        plsc.subcore_barrier()
        prow = pl.multiple_of(ss * prow_per_sub, 8)
        pltpu.sync_copy(pool_sp.at[pl.ds(prow, prow_per_sub)],
                        out_hbm.at[cc, pl.ds(prow, prow_per_sub)])

    return k


def kernel(x, edge_index, edge_attr, batch,
           W1, We1, as1, ad1, ae1, b1,
           W2, We2, as2, ad2, ae2, b2,
           W3, We3, as3, ad3, ae3, b3,
           W4, We4, as4, ad4, ae4, b4,
           Wl, bl):
    n = x.shape[0]
    e = edge_index.shape[1]
    n_pad = ((n + NWK * LANES - 1) // (NWK * LANES)) * (NWK * LANES)
    e2p = e + n_pad
    assert e % (NWK * 400) == 0 and e2p % (NWK * 240) == 0

    src, dst = edge_index[0], edge_index[1]
    x_pad = jnp.pad(x, ((0, n_pad - n), (0, 0)))
    ii = jnp.arange(n_pad, dtype=I32)
    src2 = jnp.concatenate([src, jnp.where(ii < n, ii, 0)])
    dst2 = jnp.concatenate([dst, jnp.where(ii < n, ii, n_pad)])
    batch_pad = jnp.pad(batch, (0, n_pad - n), constant_values=NGRP)

    layers = [(W1, We1, as1, ad1, ae1, b1), (W2, We2, as2, ad2, ae2, b2),
              (W3, We3, as3, ad3, ae3, b3), (W4, We4, as4, ad4, ae4, b4)]

    ae_all = jnp.concatenate(
        [jnp.einsum('dhc,hc->dh', We.reshape(DE, H, HCDIM), ae)
         for (_, We, _, _, ae, _) in layers], axis=1)
    wcats = []
    for (W, _, a_s, a_d, _, _) in layers:
        k_in = W.shape[0]
        wa_s = jnp.einsum('ihc,hc->ih', W.reshape(k_in, H, HCDIM), a_s)
        wa_d = jnp.einsum('ihc,hc->ih', W.reshape(k_in, H, HCDIM), a_d)
        wcats.append(jnp.concatenate(
            [W, wa_s, wa_d, jnp.zeros((k_in, 1152 - HHC - 16), F32)], axis=1))

    qs = _ealpha_call(edge_attr, ae_all)
    p01 = _make_pass0(n, n_pad, e)(edge_attr.reshape(-1), dst)
    qls = _qloop_call(p01[0].reshape(n_pad, 32), p01[1].reshape(n_pad, 32),
                      ae_all, n_pad)

    pass1s = [_make_pass1(n, n_pad, e, e2p) for _ in range(4)]
    pass2 = _make_pass2(n_pad, e2p)

    agg = x_pad
    for li in range(4):
        bias_prev = (jnp.zeros((1, HHC), F32) if li == 0
                     else layers[li - 1][5].reshape(1, HHC))
        outs = _proj_call(agg, wcats[li], bias_prev, li > 0, n_pad)
        hs, st128, st16 = outs[:H], outs[H], outs[H + 1]
        p1out = pass1s[li](st128, st16.reshape(-1), qs[li].reshape(-1),
                           qls[li].reshape(-1), src, dst)
        wts, denp = p1out[:H], p1out[H]
        agg = pass2(*hs, src2, dst2, *wts, denp)

    pool = _make_pass3(n, n_pad)(agg, b4, Wl[:, 0], batch_pad)
    p = pool[0] + pool[1]
    return p[:NGRP, 0] / jnp.maximum(p[:NGRP, 1], 1.0) + bl[0]


# speedup vs baseline: 13.2699x; 1.2500x over previous
import functools

import jax
import jax.numpy as jnp
from jax import lax
from jax.experimental import pallas as pl
from jax.experimental.pallas import tpu as pltpu
from jax.experimental.pallas import tpu_sc as plsc

H = 8
HCDIM = 128
HHC = 1024
DE = 16
NGRP = 64
NCORE = 2
NSUB = 16
NWK = NCORE * NSUB
LANES = 16

F32 = jnp.float32
I32 = jnp.int32


def _elu(v):
    return jnp.where(v > 0, v, jnp.exp(v) - 1.0)


def _proj_call(u, wcat, bvec, apply_act, n_pad):
    bn = 512
    k = u.shape[1]

    def body(u_ref, w_ref, b_ref, *outs):
        uu = u_ref[...]
        if apply_act:
            uu = _elu(uu + b_ref[...])
        hs = lax.dot_general(uu, w_ref[...], (((1,), (0,)), ((), ())),
                             preferred_element_type=F32)
        for i in range(H):
            outs[i][...] = hs[:, HCDIM * i:HCDIM * (i + 1)]
        st = hs[:, HHC:HHC + 16]
        outs[H][...] = jnp.concatenate(
            [st, jnp.zeros((st.shape[0], HCDIM - 16), F32)], axis=1)
        outs[H + 1][...] = st

    grid = n_pad // bn
    out_shapes = [jax.ShapeDtypeStruct((n_pad, HCDIM), F32) for _ in range(H)]
    out_shapes.append(jax.ShapeDtypeStruct((n_pad, HCDIM), F32))
    out_shapes.append(jax.ShapeDtypeStruct((n_pad, 16), F32))
    out_specs = [pl.BlockSpec((bn, HCDIM), lambda j: (j, 0))
                 for _ in range(H + 1)]
    out_specs.append(pl.BlockSpec((bn, 16), lambda j: (j, 0)))
    return pl.pallas_call(
        body,
        grid=(grid,),
        in_specs=[
            pl.BlockSpec((bn, k), lambda j: (j, 0)),
            pl.BlockSpec((k, 1152), lambda j: (0, 0)),
            pl.BlockSpec((1, HHC), lambda j: (0, 0)),
        ],
        out_specs=out_specs,
        out_shape=out_shapes,
    )(u, wcat, bvec)


def _ealpha_call(ea, ae_all):
    e = ea.shape[0]
    be = 3200

    def body(ea_ref, ae_ref, *outs):
        q = lax.dot_general(ea_ref[...], ae_ref[...], (((1,), (0,)), ((), ())),
                            preferred_element_type=F32)
        for i in range(4):
            outs[i][...] = q[:, 8 * i:8 * (i + 1)]

    return pl.pallas_call(
        body,
        grid=(e // be,),
        in_specs=[
            pl.BlockSpec((be, DE), lambda j: (j, 0)),
            pl.BlockSpec((DE, 32), lambda j: (0, 0)),
        ],
        out_specs=[pl.BlockSpec((be, 8), lambda j: (j, 0)) for _ in range(4)],
        out_shape=[jax.ShapeDtypeStruct((e, 8), F32) for _ in range(4)],
    )(ea, ae_all)


def _qloop_call(p0, p1, ae_all, n_pad):
    bn = 1280

    def body(p0_ref, p1_ref, ae_ref, *outs):
        u = p0_ref[...] + p1_ref[...]
        deg = jnp.maximum(u[:, 16:17], 1.0)
        s16 = u[:, :16] / deg
        q = lax.dot_general(s16, ae_ref[...], (((1,), (0,)), ((), ())),
                            preferred_element_type=F32)
        for i in range(4):
            outs[i][...] = q[:, 8 * i:8 * (i + 1)]

    return pl.pallas_call(
        body,
        grid=(n_pad // bn,),
        in_specs=[
            pl.BlockSpec((bn, 32), lambda j: (j, 0)),
            pl.BlockSpec((bn, 32), lambda j: (j, 0)),
            pl.BlockSpec((DE, 32), lambda j: (0, 0)),
        ],
        out_specs=[pl.BlockSpec((bn, 8), lambda j: (j, 0)) for _ in range(4)],
        out_shape=[jax.ShapeDtypeStruct((n_pad, 8), F32) for _ in range(4)],
    )(p0, p1, ae_all)


def _mesh():
    return plsc.VectorSubcoreMesh(core_axis_name="c", subcore_axis_name="s")


def _wid(cc, ss):
    return ss * NCORE + cc


IOTA = lambda: lax.iota(I32, LANES)


def _make_pass0(n, n_pad, e):
    c0 = 400
    per_w = e // NWK
    nch = per_w // c0
    ndp = n_pad // 4
    rows_per_sub = ndp // NSUB

    @functools.partial(
        pl.kernel, mesh=_mesh(),
        compiler_params=pltpu.CompilerParams(needs_layout_passes=False),
        out_type=jax.ShapeDtypeStruct((2, ndp, 128), F32),
        scratch_types=[
            pltpu.VMEM_SHARED((ndp, 128), F32),
            pltpu.VMEM((c0 + 16,), I32),
            pltpu.VMEM((c0,), I32),
            pltpu.VMEM((c0, 128), F32),
            pltpu.VMEM((c0 * DE,), F32),
        ],
    )
    def k(ea_hbm, dst_hbm, out_hbm, acc_sp, didx, d4, pay_v, ea_v):
        cc = lax.axis_index("c")
        ss = lax.axis_index("s")
        wid = _wid(cc, ss)
        zv16 = jnp.zeros((16,), F32)
        onev = jnp.where(IOTA() == 0, 1.0, 0.0).astype(F32)

        def zp(i, _):
            for j in range(8):
                pay_v[i, pl.ds(j * 16, 16)] = zv16
            return 0
        lax.fori_loop(0, c0, zp, 0)

        for r0 in range(0, rows_per_sub, c0):
            rr = min(c0, rows_per_sub - r0)
            pltpu.sync_copy(
                pay_v.at[pl.ds(0, rr)],
                acc_sp.at[pl.ds(pl.multiple_of(ss * rows_per_sub + r0, 8),
                                rr)])
        plsc.subcore_barrier()

        def chunk(ci, _):
            base = wid * per_w + ci * c0
            pltpu.sync_copy(dst_hbm.at[pl.ds(base, c0)],
                            didx.at[pl.ds(0, c0)])
            pltpu.sync_copy(ea_hbm.at[pl.ds(base * DE, c0 * DE)], ea_v)

            def grp(g, _):
                dv = didx[pl.ds(g * LANES, LANES)]
                d4[pl.ds(g * LANES, LANES)] = lax.shift_right_logical(dv, 2)
                return 0
            lax.fori_loop(0, c0 // LANES, grp, 0)

            def ed(i, _):
                de = didx[pl.ds(i, 16)][0]
                col = (de & 3) * 32
                pay_v[i, pl.ds(col, 16)] = ea_v[pl.ds(i * DE, 16)]
                pay_v[i, pl.ds(col + 16, 16)] = onev
                return 0
            lax.fori_loop(0, c0, ed, 0)

            pltpu.sync_copy(pay_v, acc_sp.at[d4], add=True)

            def ed2(i, _):
                de = didx[pl.ds(i, 16)][0]
                col = (de & 3) * 32
                pay_v[i, pl.ds(col, 16)] = zv16
                pay_v[i, pl.ds(col + 16, 16)] = zv16
                return 0
            lax.fori_loop(0, c0, ed2, 0)
            return 0
        lax.fori_loop(0, nch, chunk, 0)

        plsc.subcore_barrier()
        row = pl.multiple_of(ss * rows_per_sub, 8)
        pltpu.sync_copy(acc_sp.at[pl.ds(row, rows_per_sub)],
                        out_hbm.at[cc, pl.ds(row, rows_per_sub)])

    return k


def _make_pass1(n, n_pad, e, e2p):
    c1 = 80
    per_w = e // NWK
    nch = per_w // c1
    ngrp = c1 // LANES
    ndp = n_pad // 8 + 128
    den_rows = ndp // NSUB
    pslf = n_pad // NWK

    @functools.partial(
        pl.kernel, mesh=_mesh(),
        compiler_params=pltpu.CompilerParams(needs_layout_passes=False),
        out_type=[
            jax.ShapeDtypeStruct((H * e2p,), F32),
            jax.ShapeDtypeStruct((2, ndp, 128), F32),
        ],
        scratch_types=[
            pltpu.VMEM_SHARED((ndp, 128), F32),
            pltpu.VMEM((c1,), I32),
            pltpu.VMEM((c1,), I32),
            pltpu.VMEM((c1,), I32),
            pltpu.VMEM((c1, 128), F32),
            pltpu.VMEM((c1, 128), F32),
            pltpu.VMEM((c1 * 8,), F32),
            pltpu.VMEM((H, c1), F32),
            pltpu.VMEM((c1, 128), F32),
            pltpu.VMEM((den_rows, 128), F32),
            pltpu.VMEM((pslf * 16,), F32),
            pltpu.VMEM((pslf * 8,), F32),
        ],
    )
    def k(st128_hbm, st16_hbm, q_hbm, ql_hbm, src_hbm, dst_hbm,
          w_hbm, den_hbm,
          den_sp, sidx, didx, d8, sbuf, tbuf, qbuf, wbuf, pay_v, z_v,
          st16buf, qlbuf):
        cc = lax.axis_index("c")
        ss = lax.axis_index("s")
        wid = _wid(cc, ss)
        zv16 = jnp.zeros((16,), F32)

        def zz(i, _):
            for j in range(8):
                z_v[i, pl.ds(j * 16, 16)] = zv16
            return 0
        lax.fori_loop(0, den_rows, zz, 0)

        def zp(i, _):
            for j in range(8):
                pay_v[i, pl.ds(j * 16, 16)] = zv16
            return 0
        lax.fori_loop(0, c1, zp, 0)

        pltpu.sync_copy(
            z_v, den_sp.at[pl.ds(pl.multiple_of(ss * den_rows, 8),
                                 den_rows)])
        plsc.subcore_barrier()

        def chunk(ci, _):
            base = wid * per_w + ci * c1
            pltpu.sync_copy(src_hbm.at[pl.ds(base, c1)], sidx)
            pltpu.sync_copy(dst_hbm.at[pl.ds(base, c1)], didx)
            pltpu.sync_copy(q_hbm.at[pl.ds(base * 8, c1 * 8)], qbuf)
            pltpu.sync_copy(st128_hbm.at[sidx], sbuf)
            pltpu.sync_copy(st128_hbm.at[didx], tbuf)

            def grp(g, _):
                rows = g * LANES + IOTA()
                dv = didx[pl.ds(g * LANES, LANES)]
                d8[pl.ds(g * LANES, LANES)] = lax.shift_right_logical(dv, 3)
                colv = (dv & 7) * 8
                for hh in range(H):
                    colh = jnp.full((LANES,), hh, I32)
                    sv = plsc.load_gather(sbuf, [rows, colh])
                    tv = plsc.load_gather(tbuf, [rows, colh + 8])
                    qv = plsc.load_gather(qbuf, [rows * 8 + hh])
                    al = sv + tv + qv
                    al = jnp.maximum(al, 0.2 * al)
                    wv = jnp.exp(al)
                    wbuf[hh, pl.ds(g * LANES, LANES)] = wv
                    plsc.store_scatter(pay_v, [rows, colv + hh], wv)
                return 0
            lax.fori_loop(0, ngrp, grp, 0)

            for hh in range(H):
                pltpu.sync_copy(wbuf.at[hh],
                                w_hbm.at[pl.ds(hh * e2p + base, c1)])
            pltpu.sync_copy(pay_v, den_sp.at[d8], add=True)

            def clr(g, _):
                rows = g * LANES + IOTA()
                dv = didx[pl.ds(g * LANES, LANES)]
                colv = (dv & 7) * 8
                for hh in range(H):
                    plsc.store_scatter(pay_v, [rows, colv + hh],
                                       jnp.zeros((LANES,), F32))
                return 0
            lax.fori_loop(0, ngrp, clr, 0)
            return 0
        lax.fori_loop(0, nch, chunk, 0)

        nbase = wid * pslf
        pltpu.sync_copy(st16_hbm.at[pl.ds(nbase * 16, pslf * 16)], st16buf)
        pltpu.sync_copy(ql_hbm.at[pl.ds(nbase * 8, pslf * 8)], qlbuf)

        for sb in range(pslf // c1):
            def sgrp(g, _):
                rows = sb * c1 + g * LANES + IOTA()
                node = nbase + rows
                valid = node < n
                prow = g * LANES + IOTA()
                d8[pl.ds(g * LANES, LANES)] = lax.shift_right_logical(node, 3)
                colv = (node & 7) * 8
                for hh in range(H):
                    sv = plsc.load_gather(st16buf, [rows * 16 + hh])
                    tv = plsc.load_gather(st16buf, [rows * 16 + 8 + hh])
                    qv = plsc.load_gather(qlbuf, [rows * 8 + hh])
                    al = sv + tv + qv
                    al = jnp.maximum(al, 0.2 * al)
                    wv = jnp.where(valid, jnp.exp(al), 0.0)
                    wbuf[hh, pl.ds(g * LANES, LANES)] = wv
                    plsc.store_scatter(pay_v, [prow, colv + hh], wv)
                return 0
            lax.fori_loop(0, ngrp, sgrp, 0)

            for hh in range(H):
                pltpu.sync_copy(
                    wbuf.at[hh],
                    w_hbm.at[pl.ds(hh * e2p + e + nbase + sb * c1, c1)])
            pltpu.sync_copy(pay_v, den_sp.at[d8], add=True)

            def sclr(g, _):
                rows = sb * c1 + g * LANES + IOTA()
                node = nbase + rows
                prow = g * LANES + IOTA()
                colv = (node & 7) * 8
                for hh in range(H):
                    plsc.store_scatter(pay_v, [prow, colv + hh],
                                       jnp.zeros((LANES,), F32))
                return 0
            lax.fori_loop(0, ngrp, sclr, 0)

        plsc.subcore_barrier()
        drow = pl.multiple_of(ss * den_rows, 8)
        pltpu.sync_copy(den_sp.at[pl.ds(drow, den_rows)],
                        den_hbm.at[cc, pl.ds(drow, den_rows)])

    return k


def _make_pass2(n_pad, e2p):
    c2 = 120
    per_t = e2p // NSUB
    nch = per_t // c2
    na = n_pad + 16
    rb = 64
    rows_per_sub = n_pad // NSUB

    @functools.partial(
        pl.kernel, mesh=_mesh(),
        compiler_params=pltpu.CompilerParams(needs_layout_passes=False),
        out_type=jax.ShapeDtypeStruct((n_pad, HHC), F32),
        scratch_types=[
            pltpu.VMEM_SHARED((na, HCDIM), F32),
            pltpu.VMEM((c2,), I32),
            pltpu.VMEM((c2,), I32),
            pltpu.VMEM((c2,), I32),
            pltpu.VMEM((c2,), I32),
            pltpu.VMEM((c2 + 16, ), F32),
            pltpu.VMEM((c2 + 16, ), F32),
            pltpu.VMEM((c2, HCDIM), F32),
            pltpu.VMEM((c2, HCDIM), F32),
            pltpu.VMEM((rb, HCDIM), F32),
            pltpu.VMEM((rb // 8, 128), F32),
            pltpu.VMEM((rb // 8, 128), F32),
            pltpu.SemaphoreType.DMA,
            pltpu.SemaphoreType.DMA,
            pltpu.SemaphoreType.DMA,
            pltpu.SemaphoreType.DMA,
        ],
    )
    def k(h3_hbm, src_hbm, dst_hbm, w_hbm, den_hbm,
          out_hbm, acc_sp, sidx0, sidx1, didx0, didx1, wv0, wv1,
          rows0, rows1, eb_v, d0_v, d1_v, gsem0, gsem1, ssem0, ssem1):
        cc = lax.axis_index("c")
        ss = lax.axis_index("s")
        sidxs, didxs = [sidx0, sidx1], [didx0, didx1]
        wvs, rowss = [wv0, wv1], [rows0, rows1]
        gsems, ssems = [gsem0, gsem1], [ssem0, ssem1]
        zv16 = jnp.zeros((16,), F32)

        def headloop(hi, _):
            head = cc * 4 + hi
            htab = h3_hbm.at[head]

            def zz(i, _):
                for j in range(HCDIM // LANES):
                    eb_v[i, pl.ds(j * LANES, LANES)] = zv16
                return 0
            lax.fori_loop(0, rb, zz, 0)
            for r0 in range(0, rows_per_sub, rb):
                pltpu.sync_copy(
                    eb_v,
                    acc_sp.at[pl.ds(
                        pl.multiple_of(ss * rows_per_sub + r0, 8), rb)])

            @pl.when(ss == 0)
            def _():
                pltpu.sync_copy(eb_v.at[pl.ds(0, 16)],
                                acc_sp.at[pl.ds(n_pad, 16)])
            plsc.subcore_barrier()

            def load_idx(ci, b):
                base = ss * per_t + ci * c2
                pltpu.sync_copy(src_hbm.at[pl.ds(base, c2)], sidxs[b])
                pltpu.sync_copy(dst_hbm.at[pl.ds(base, c2)], didxs[b])
                pltpu.sync_copy(w_hbm.at[pl.ds(head * e2p + base, c2)],
                                wvs[b].at[pl.ds(0, c2)])

            load_idx(0, 0)
            pltpu.async_copy(htab.at[sidxs[0]], rows0, gsem0)

            def pair(pi, _):
                for b in (0, 1):
                    ci = 2 * pi + b
                    o = 1 - b
                    pltpu.make_async_copy(
                        htab.at[sidxs[b]], rowss[b], gsems[b]).wait()

                    @pl.when(ci + 1 < nch)
                    def _():
                        @pl.when(ci >= 1)
                        def _():
                            pltpu.make_async_copy(
                                rowss[o], acc_sp.at[didxs[o]],
                                ssems[o]).wait()
                        load_idx(ci + 1, o)
                        pltpu.async_copy(htab.at[sidxs[o]], rowss[o],
                                         gsems[o])

                    def edge(ei, _):
                        we = wvs[b][pl.ds(ei, 16)][0]
                        for j in range(HCDIM // LANES):
                            v = rowss[b][ei, pl.ds(j * LANES, LANES)]
                            rowss[b][ei, pl.ds(j * LANES, LANES)] = v * we
                        return 0
                    lax.fori_loop(0, c2, edge, 0)

                    pltpu.async_copy(rowss[b], acc_sp.at[didxs[b]],
                                     ssems[b], add=True)
                return 0
            lax.fori_loop(0, nch // 2, pair, 0)
            pltpu.make_async_copy(rows0, acc_sp.at[didx0], ssem0).wait()
            pltpu.make_async_copy(rows1, acc_sp.at[didx1], ssem1).wait()
            plsc.subcore_barrier()

            for r0 in range(0, rows_per_sub, rb):
                row0 = pl.multiple_of(ss * rows_per_sub + r0, 8)
                row8 = pl.multiple_of((ss * rows_per_sub + r0) // 8, 8)
                pltpu.sync_copy(acc_sp.at[pl.ds(row0, rb)], eb_v)
                pltpu.sync_copy(den_hbm.at[0, pl.ds(row8, rb // 8)], d0_v)
                pltpu.sync_copy(den_hbm.at[1, pl.ds(row8, rb // 8)], d1_v)

                def erow(i, _):
                    pr = lax.shift_right_logical(i, 3)
                    pc = (i & 7) * 8
                    d0r = d0_v[pr, pl.ds(pc, 16)]
                    d1r = d1_v[pr, pl.ds(pc, 16)]
                    invv = 1.0 / (d0r + d1r + 1e-16)
                    inv = jnp.sum(jnp.where(IOTA() == head, invv, 0.0))
                    for j in range(HCDIM // LANES):
                        v = eb_v[i, pl.ds(j * LANES, LANES)]
                        eb_v[i, pl.ds(j * LANES, LANES)] = v * inv
                    return 0
                lax.fori_loop(0, rb, erow, 0)

                pltpu.sync_copy(
                    eb_v,
                    out_hbm.at[pl.ds(row0, rb),
                               pl.ds(pl.multiple_of(head * HCDIM, 128),
                                     HCDIM)])
            plsc.subcore_barrier()
            return 0
        lax.fori_loop(0, H // 2, headloop, 0)

    return k


def _make_pass3(n, n_pad):
    pslf = n_pad // NWK
    sbn = 80
    cn = 8
    prows = 128
    prow_per_sub = prows // NSUB

    @functools.partial(
        pl.kernel, mesh=_mesh(),
        compiler_params=pltpu.CompilerParams(needs_layout_passes=False),
        out_type=jax.ShapeDtypeStruct((2, prows, 128), F32),
        scratch_types=[
            pltpu.VMEM_SHARED((prows, 128), F32),
            pltpu.VMEM((cn, HHC), F32),
            pltpu.VMEM((sbn, 128), F32),
            pltpu.VMEM((sbn,), I32),
            pltpu.VMEM((HCDIM,), F32),
            pltpu.VMEM((HCDIM,), F32),
        ],
    )
    def k(agg_hbm, b_hbm, wl_hbm, batch_hbm, out_hbm,
          pool_sp, ab_v, pay_v, bidx, b4_v, wl_v):
        cc = lax.axis_index("c")
        ss = lax.axis_index("s")
        wid = _wid(cc, ss)
        nbase = wid * pslf
        zv16 = jnp.zeros((16,), F32)

        def zp(i, _):
            for j in range(8):
                pay_v[i, pl.ds(j * 16, 16)] = zv16
            return 0
        lax.fori_loop(0, sbn, zp, 0)

        pltpu.sync_copy(
            pay_v.at[pl.ds(0, prow_per_sub)],
            pool_sp.at[pl.ds(pl.multiple_of(ss * prow_per_sub, 8),
                             prow_per_sub)])
        pltpu.sync_copy(b_hbm, b4_v)
        pltpu.sync_copy(wl_hbm, wl_v)
        plsc.subcore_barrier()

        for sb in range(pslf // sbn):
            sb_base = nbase + sb * sbn
            pltpu.sync_copy(batch_hbm.at[pl.ds(sb_base, sbn)], bidx)

            def chunk(ci, _):
                pltpu.sync_copy(agg_hbm.at[pl.ds(sb_base + ci * cn, cn)],
                                ab_v)

                def node(i, _):
                    zacc = 0.0
                    for j in range(HCDIM // LANES):
                        acc = ab_v[i, pl.ds(j * LANES, LANES)]
                        for hh in range(1, H):
                            acc = acc + ab_v[i, pl.ds(hh * HCDIM + j * LANES,
                                                      LANES)]
                        o = acc * (1.0 / H) + b4_v[pl.ds(j * LANES, LANES)]
                        o = jnp.where(o > 0, o, jnp.exp(o) - 1.0)
                        zacc = zacc + jnp.sum(o * wl_v[pl.ds(j * LANES,
                                                             LANES)])
                    node_id = sb_base + ci * cn + i
                    validf = jnp.where(node_id < n, 1.0, 0.0)
                    lane = IOTA()
                    row = (jnp.where(lane == 0, zacc * validf, 0.0)
                           + jnp.where(lane == 1, validf, 0.0))
                    pay_v[ci * cn + i, pl.ds(0, 16)] = row.astype(F32)
                    return 0
                lax.fori_loop(0, cn, node, 0)
                return 0
            lax.fori_loop(0, sbn // cn, chunk, 0)

            pltpu.sync_copy(pay_v, pool_sp.at[bidx], add=True)
        plsc.subcore_barrier()
        prow = pl.multiple_of(ss * prow_per_sub, 8)
        pltpu.sync_copy(pool_sp.at[pl.ds(prow, prow_per_sub)],
                        out_hbm.at[cc, pl.ds(prow, prow_per_sub)])

    return k


def kernel(x, edge_index, edge_attr, batch,
           W1, We1, as1, ad1, ae1, b1,
           W2, We2, as2, ad2, ae2, b2,
           W3, We3, as3, ad3, ae3, b3,
           W4, We4, as4, ad4, ae4, b4,
           Wl, bl):
    n = x.shape[0]
    e = edge_index.shape[1]
    n_pad = ((n + NWK * LANES - 1) // (NWK * LANES)) * (NWK * LANES)
    e2p = e + n_pad
    assert e % (NWK * 400) == 0 and e2p % (NWK * 240) == 0

    src, dst = edge_index[0], edge_index[1]
    x_pad = jnp.pad(x, ((0, n_pad - n), (0, 0)))
    ii = jnp.arange(n_pad, dtype=I32)
    src2 = jnp.concatenate([src, jnp.where(ii < n, ii, 0)])
    dst2 = jnp.concatenate([dst, jnp.where(ii < n, ii, n_pad)])
    batch_pad = jnp.pad(batch, (0, n_pad - n), constant_values=NGRP)

    layers = [(W1, We1, as1, ad1, ae1, b1), (W2, We2, as2, ad2, ae2, b2),
              (W3, We3, as3, ad3, ae3, b3), (W4, We4, as4, ad4, ae4, b4)]

    ae_all = jnp.concatenate(
        [jnp.einsum('dhc,hc->dh', We.reshape(DE, H, HCDIM), ae)
         for (_, We, _, _, ae, _) in layers], axis=1)
    wcats = []
    for (W, _, a_s, a_d, _, _) in layers:
        k_in = W.shape[0]
        wa_s = jnp.einsum('ihc,hc->ih', W.reshape(k_in, H, HCDIM), a_s)
        wa_d = jnp.einsum('ihc,hc->ih', W.reshape(k_in, H, HCDIM), a_d)
        wcats.append(jnp.concatenate(
            [W, wa_s, wa_d, jnp.zeros((k_in, 1152 - HHC - 16), F32)], axis=1))

    qs = _ealpha_call(edge_attr, ae_all)
    p01 = _make_pass0(n, n_pad, e)(edge_attr.reshape(-1), dst)
    qls = _qloop_call(p01[0].reshape(n_pad, 32), p01[1].reshape(n_pad, 32),
                      ae_all, n_pad)

    pass1s = [_make_pass1(n, n_pad, e, e2p) for _ in range(4)]
    pass2 = _make_pass2(n_pad, e2p)

    agg = x_pad
    for li in range(4):
        bias_prev = (jnp.zeros((1, HHC), F32) if li == 0
                     else layers[li - 1][5].reshape(1, HHC))
        outs = _proj_call(agg, wcats[li], bias_prev, li > 0, n_pad)
        hs, st128, st16 = outs[:H], outs[H], outs[H + 1]
        w_flat, denp = pass1s[li](st128, st16.reshape(-1),
                                  qs[li].reshape(-1),
                                  qls[li].reshape(-1), src, dst)
        h3 = jnp.stack(hs)
        agg = pass2(h3, src2, dst2, w_flat, denp)

    pool = _make_pass3(n, n_pad)(agg, b4, Wl[:, 0], batch_pad)
    p = pool[0] + pool[1]
    return p[:NGRP, 0] / jnp.maximum(p[:NGRP, 1], 1.0) + bl[0]


# speedup vs baseline: 13.5983x; 1.0247x over previous
import functools

import jax
import jax.numpy as jnp
from jax import lax
from jax.experimental import pallas as pl
from jax.experimental.pallas import tpu as pltpu
from jax.experimental.pallas import tpu_sc as plsc

H = 8
HCDIM = 128
HHC = 1024
DE = 16
NGRP = 64
NCORE = 2
NSUB = 16
NWK = NCORE * NSUB
LANES = 16

F32 = jnp.float32
I32 = jnp.int32


def _elu(v):
    return jnp.where(v > 0, v, jnp.exp(v) - 1.0)


def _proj_call(u, wcat, bvec, apply_act, n_pad):
    bn = 512
    k = u.shape[1]

    def body(u_ref, w_ref, b_ref, *outs):
        uu = u_ref[...]
        if apply_act:
            uu = _elu(uu + b_ref[...])
        hs = lax.dot_general(uu, w_ref[...], (((1,), (0,)), ((), ())),
                             preferred_element_type=F32)
        for i in range(H):
            outs[i][...] = hs[:, HCDIM * i:HCDIM * (i + 1)]
        st = hs[:, HHC:HHC + 16]
        outs[H][...] = jnp.concatenate(
            [st, jnp.zeros((st.shape[0], HCDIM - 16), F32)], axis=1)
        outs[H + 1][...] = st

    grid = n_pad // bn
    out_shapes = [jax.ShapeDtypeStruct((n_pad, HCDIM), F32) for _ in range(H)]
    out_shapes.append(jax.ShapeDtypeStruct((n_pad, HCDIM), F32))
    out_shapes.append(jax.ShapeDtypeStruct((n_pad, 16), F32))
    out_specs = [pl.BlockSpec((bn, HCDIM), lambda j: (j, 0))
                 for _ in range(H + 1)]
    out_specs.append(pl.BlockSpec((bn, 16), lambda j: (j, 0)))
    return pl.pallas_call(
        body,
        grid=(grid,),
        in_specs=[
            pl.BlockSpec((bn, k), lambda j: (j, 0)),
            pl.BlockSpec((k, 1152), lambda j: (0, 0)),
            pl.BlockSpec((1, HHC), lambda j: (0, 0)),
        ],
        out_specs=out_specs,
        out_shape=out_shapes,
    )(u, wcat, bvec)


def _ealpha_call(ea, ae_all):
    e = ea.shape[0]
    be = 3200

    def body(ea_ref, ae_ref, *outs):
        q = lax.dot_general(ea_ref[...], ae_ref[...], (((1,), (0,)), ((), ())),
                            preferred_element_type=F32)
        for i in range(4):
            outs[i][...] = q[:, 8 * i:8 * (i + 1)]

    return pl.pallas_call(
        body,
        grid=(e // be,),
        in_specs=[
            pl.BlockSpec((be, DE), lambda j: (j, 0)),
            pl.BlockSpec((DE, 32), lambda j: (0, 0)),
        ],
        out_specs=[pl.BlockSpec((be, 8), lambda j: (j, 0)) for _ in range(4)],
        out_shape=[jax.ShapeDtypeStruct((e, 8), F32) for _ in range(4)],
    )(ea, ae_all)


def _qloop_call(p0, p1, ae_all, n_pad):
    bn = 1280

    def body(p0_ref, p1_ref, ae_ref, *outs):
        u = p0_ref[...] + p1_ref[...]
        deg = jnp.maximum(u[:, 16:17], 1.0)
        s16 = u[:, :16] / deg
        q = lax.dot_general(s16, ae_ref[...], (((1,), (0,)), ((), ())),
                            preferred_element_type=F32)
        for i in range(4):
            outs[i][...] = q[:, 8 * i:8 * (i + 1)]

    return pl.pallas_call(
        body,
        grid=(n_pad // bn,),
        in_specs=[
            pl.BlockSpec((bn, 32), lambda j: (j, 0)),
            pl.BlockSpec((bn, 32), lambda j: (j, 0)),
            pl.BlockSpec((DE, 32), lambda j: (0, 0)),
        ],
        out_specs=[pl.BlockSpec((bn, 8), lambda j: (j, 0)) for _ in range(4)],
        out_shape=[jax.ShapeDtypeStruct((n_pad, 8), F32) for _ in range(4)],
    )(p0, p1, ae_all)


def _mesh():
    return plsc.VectorSubcoreMesh(core_axis_name="c", subcore_axis_name="s")


def _wid(cc, ss):
    return ss * NCORE + cc


IOTA = lambda: lax.iota(I32, LANES)


def _make_pass0(n, n_pad, e):
    c0 = 400
    per_w = e // NWK
    nch = per_w // c0
    ndp = n_pad // 4
    rows_per_sub = ndp // NSUB

    @functools.partial(
        pl.kernel, mesh=_mesh(),
        compiler_params=pltpu.CompilerParams(needs_layout_passes=False),
        out_type=jax.ShapeDtypeStruct((2, ndp, 128), F32),
        scratch_types=[
            pltpu.VMEM_SHARED((ndp, 128), F32),
            pltpu.VMEM((c0 + 16,), I32),
            pltpu.VMEM((c0,), I32),
            pltpu.VMEM((c0, 128), F32),
            pltpu.VMEM((c0 * DE,), F32),
        ],
    )
    def k(ea_hbm, dst_hbm, out_hbm, acc_sp, didx, d4, pay_v, ea_v):
        cc = lax.axis_index("c")
        ss = lax.axis_index("s")
        wid = _wid(cc, ss)
        zv16 = jnp.zeros((16,), F32)
        onev = jnp.where(IOTA() == 0, 1.0, 0.0).astype(F32)

        def zp(i, _):
            for j in range(8):
                pay_v[i, pl.ds(j * 16, 16)] = zv16
            return 0
        lax.fori_loop(0, c0, zp, 0)

        for r0 in range(0, rows_per_sub, c0):
            rr = min(c0, rows_per_sub - r0)
            pltpu.sync_copy(
                pay_v.at[pl.ds(0, rr)],
                acc_sp.at[pl.ds(pl.multiple_of(ss * rows_per_sub + r0, 8),
                                rr)])
        plsc.subcore_barrier()

        def chunk(ci, _):
            base = wid * per_w + ci * c0
            pltpu.sync_copy(dst_hbm.at[pl.ds(base, c0)],
                            didx.at[pl.ds(0, c0)])
            pltpu.sync_copy(ea_hbm.at[pl.ds(base * DE, c0 * DE)], ea_v)

            def grp(g, _):
                dv = didx[pl.ds(g * LANES, LANES)]
                d4[pl.ds(g * LANES, LANES)] = lax.shift_right_logical(dv, 2)
                return 0
            lax.fori_loop(0, c0 // LANES, grp, 0)

            def ed(i, _):
                de = didx[pl.ds(i, 16)][0]
                col = (de & 3) * 32
                pay_v[i, pl.ds(col, 16)] = ea_v[pl.ds(i * DE, 16)]
                pay_v[i, pl.ds(col + 16, 16)] = onev
                return 0
            lax.fori_loop(0, c0, ed, 0)

            pltpu.sync_copy(pay_v, acc_sp.at[d4], add=True)

            def ed2(i, _):
                de = didx[pl.ds(i, 16)][0]
                col = (de & 3) * 32
                pay_v[i, pl.ds(col, 16)] = zv16
                pay_v[i, pl.ds(col + 16, 16)] = zv16
                return 0
            lax.fori_loop(0, c0, ed2, 0)
            return 0
        lax.fori_loop(0, nch, chunk, 0)

        plsc.subcore_barrier()
        row = pl.multiple_of(ss * rows_per_sub, 8)
        pltpu.sync_copy(acc_sp.at[pl.ds(row, rows_per_sub)],
                        out_hbm.at[cc, pl.ds(row, rows_per_sub)])

    return k


def _make_pass1(n, n_pad, e, e2p):
    c1 = 80
    per_w = e // NWK
    nch = per_w // c1
    ngrp = c1 // LANES
    ndp = n_pad // 8 + 128
    den_rows = ndp // NSUB
    pslf = n_pad // NWK

    @functools.partial(
        pl.kernel, mesh=_mesh(),
        compiler_params=pltpu.CompilerParams(needs_layout_passes=False),
        out_type=[
            jax.ShapeDtypeStruct((H * e2p,), F32),
            jax.ShapeDtypeStruct((2, ndp, 128), F32),
        ],
        scratch_types=[
            pltpu.VMEM_SHARED((ndp, 128), F32),
            pltpu.VMEM((c1,), I32),
            pltpu.VMEM((c1,), I32),
            pltpu.VMEM((c1,), I32),
            pltpu.VMEM((c1, 128), F32),
            pltpu.VMEM((c1, 128), F32),
            pltpu.VMEM((c1 * 8,), F32),
            pltpu.VMEM((H, c1), F32),
            pltpu.VMEM((c1, 128), F32),
            pltpu.VMEM((den_rows, 128), F32),
            pltpu.VMEM((pslf * 16,), F32),
            pltpu.VMEM((pslf * 8,), F32),
        ],
    )
    def k(st128_hbm, st16_hbm, q_hbm, ql_hbm, src_hbm, dst_hbm,
          w_hbm, den_hbm,
          den_sp, sidx, didx, d8, sbuf, tbuf, qbuf, wbuf, pay_v, z_v,
          st16buf, qlbuf):
        cc = lax.axis_index("c")
        ss = lax.axis_index("s")
        wid = _wid(cc, ss)
        zv16 = jnp.zeros((16,), F32)

        def zz(i, _):
            for j in range(8):
                z_v[i, pl.ds(j * 16, 16)] = zv16
            return 0
        lax.fori_loop(0, den_rows, zz, 0)

        def zp(i, _):
            for j in range(8):
                pay_v[i, pl.ds(j * 16, 16)] = zv16
            return 0
        lax.fori_loop(0, c1, zp, 0)

        pltpu.sync_copy(
            z_v, den_sp.at[pl.ds(pl.multiple_of(ss * den_rows, 8),
                                 den_rows)])
        plsc.subcore_barrier()

        def chunk(ci, _):
            base = wid * per_w + ci * c1
            pltpu.sync_copy(src_hbm.at[pl.ds(base, c1)], sidx)
            pltpu.sync_copy(dst_hbm.at[pl.ds(base, c1)], didx)
            pltpu.sync_copy(q_hbm.at[pl.ds(base * 8, c1 * 8)], qbuf)
            pltpu.sync_copy(st128_hbm.at[sidx], sbuf)
            pltpu.sync_copy(st128_hbm.at[didx], tbuf)

            def grp(g, _):
                rows = g * LANES + IOTA()
                dv = didx[pl.ds(g * LANES, LANES)]
                d8[pl.ds(g * LANES, LANES)] = lax.shift_right_logical(dv, 3)
                colv = (dv & 7) * 8
                for hh in range(H):
                    colh = jnp.full((LANES,), hh, I32)
                    sv = plsc.load_gather(sbuf, [rows, colh])
                    tv = plsc.load_gather(tbuf, [rows, colh + 8])
                    qv = plsc.load_gather(qbuf, [rows * 8 + hh])
                    al = sv + tv + qv
                    al = jnp.maximum(al, 0.2 * al)
                    wv = jnp.exp(al)
                    wbuf[hh, pl.ds(g * LANES, LANES)] = wv
                    plsc.store_scatter(pay_v, [rows, colv + hh], wv)
                return 0
            lax.fori_loop(0, ngrp, grp, 0)

            for hh in range(H):
                pltpu.sync_copy(wbuf.at[hh],
                                w_hbm.at[pl.ds(hh * e2p + base, c1)])
            pltpu.sync_copy(pay_v, den_sp.at[d8], add=True)

            def clr(g, _):
                rows = g * LANES + IOTA()
                dv = didx[pl.ds(g * LANES, LANES)]
                colv = (dv & 7) * 8
                for hh in range(H):
                    plsc.store_scatter(pay_v, [rows, colv + hh],
                                       jnp.zeros((LANES,), F32))
                return 0
            lax.fori_loop(0, ngrp, clr, 0)
            return 0
        lax.fori_loop(0, nch, chunk, 0)

        nbase = wid * pslf
        pltpu.sync_copy(st16_hbm.at[pl.ds(nbase * 16, pslf * 16)], st16buf)
        pltpu.sync_copy(ql_hbm.at[pl.ds(nbase * 8, pslf * 8)], qlbuf)

        for sb in range(pslf // c1):
            def sgrp(g, _):
                rows = sb * c1 + g * LANES + IOTA()
                node = nbase + rows
                valid = node < n
                prow = g * LANES + IOTA()
                d8[pl.ds(g * LANES, LANES)] = lax.shift_right_logical(node, 3)
                colv = (node & 7) * 8
                for hh in range(H):
                    sv = plsc.load_gather(st16buf, [rows * 16 + hh])
                    tv = plsc.load_gather(st16buf, [rows * 16 + 8 + hh])
                    qv = plsc.load_gather(qlbuf, [rows * 8 + hh])
                    al = sv + tv + qv
                    al = jnp.maximum(al, 0.2 * al)
                    wv = jnp.where(valid, jnp.exp(al), 0.0)
                    wbuf[hh, pl.ds(g * LANES, LANES)] = wv
                    plsc.store_scatter(pay_v, [prow, colv + hh], wv)
                return 0
            lax.fori_loop(0, ngrp, sgrp, 0)

            for hh in range(H):
                pltpu.sync_copy(
                    wbuf.at[hh],
                    w_hbm.at[pl.ds(hh * e2p + e + nbase + sb * c1, c1)])
            pltpu.sync_copy(pay_v, den_sp.at[d8], add=True)

            def sclr(g, _):
                rows = sb * c1 + g * LANES + IOTA()
                node = nbase + rows
                prow = g * LANES + IOTA()
                colv = (node & 7) * 8
                for hh in range(H):
                    plsc.store_scatter(pay_v, [prow, colv + hh],
                                       jnp.zeros((LANES,), F32))
                return 0
            lax.fori_loop(0, ngrp, sclr, 0)

        plsc.subcore_barrier()
        drow = pl.multiple_of(ss * den_rows, 8)
        pltpu.sync_copy(den_sp.at[pl.ds(drow, den_rows)],
                        den_hbm.at[cc, pl.ds(drow, den_rows)])

    return k


def _make_pass2(n_pad, e2p):
    c2 = 120
    per_t = e2p // NSUB
    nch = per_t // c2
    na = n_pad + 16
    rb = 64
    rows_per_sub = n_pad // NSUB

    @functools.partial(
        pl.kernel, mesh=_mesh(),
        compiler_params=pltpu.CompilerParams(needs_layout_passes=False),
        out_type=jax.ShapeDtypeStruct((n_pad, HHC), F32),
        scratch_types=[
            pltpu.VMEM_SHARED((na, HCDIM), F32),
            pltpu.VMEM((c2,), I32),
            pltpu.VMEM((c2,), I32),
            pltpu.VMEM((c2,), I32),
            pltpu.VMEM((c2,), I32),
            pltpu.VMEM((c2 + 16, ), F32),
            pltpu.VMEM((c2 + 16, ), F32),
            pltpu.VMEM((c2, HCDIM), F32),
            pltpu.VMEM((c2, HCDIM), F32),
            pltpu.VMEM((rb, HCDIM), F32),
            pltpu.VMEM((rb // 8, 128), F32),
            pltpu.VMEM((rb // 8, 128), F32),
            pltpu.SemaphoreType.DMA,
            pltpu.SemaphoreType.DMA,
            pltpu.SemaphoreType.DMA,
            pltpu.SemaphoreType.DMA,
        ],
    )
    def k(h3_hbm, src_hbm, dst_hbm, w_hbm, den_hbm,
          out_hbm, acc_sp, sidx0, sidx1, didx0, didx1, wv0, wv1,
          rows0, rows1, eb_v, d0_v, d1_v, gsem0, gsem1, ssem0, ssem1):
        cc = lax.axis_index("c")
        ss = lax.axis_index("s")
        sidxs, didxs = [sidx0, sidx1], [didx0, didx1]
        wvs, rowss = [wv0, wv1], [rows0, rows1]
        gsems, ssems = [gsem0, gsem1], [ssem0, ssem1]
        zv16 = jnp.zeros((16,), F32)

        def headloop(hi, _):
            head = cc * 4 + hi
            htab = h3_hbm.at[head]

            def zz(i, _):
                for j in range(HCDIM // LANES):
                    eb_v[i, pl.ds(j * LANES, LANES)] = zv16
                return 0
            lax.fori_loop(0, rb, zz, 0)
            for r0 in range(0, rows_per_sub, rb):
                pltpu.sync_copy(
                    eb_v,
                    acc_sp.at[pl.ds(
                        pl.multiple_of(ss * rows_per_sub + r0, 8), rb)])

            @pl.when(ss == 0)
            def _():
                pltpu.sync_copy(eb_v.at[pl.ds(0, 16)],
                                acc_sp.at[pl.ds(n_pad, 16)])
            plsc.subcore_barrier()

            def load_idx(ci, b):
                base = ss * per_t + ci * c2
                pltpu.sync_copy(src_hbm.at[pl.ds(base, c2)], sidxs[b])
                pltpu.sync_copy(dst_hbm.at[pl.ds(base, c2)], didxs[b])
                pltpu.sync_copy(w_hbm.at[pl.ds(head * e2p + base, c2)],
                                wvs[b].at[pl.ds(0, c2)])

            load_idx(0, 0)
            pltpu.async_copy(htab.at[sidxs[0]], rows0, gsem0)

            def pair(pi, _):
                for b in (0, 1):
                    ci = 2 * pi + b
                    o = 1 - b
                    pltpu.make_async_copy(
                        htab.at[sidxs[b]], rowss[b], gsems[b]).wait()

                    @pl.when(ci + 1 < nch)
                    def _():
                        @pl.when(ci >= 1)
                        def _():
                            pltpu.make_async_copy(
                                rowss[o], acc_sp.at[didxs[o]],
                                ssems[o]).wait()
                        load_idx(ci + 1, o)
                        pltpu.async_copy(htab.at[sidxs[o]], rowss[o],
                                         gsems[o])

                    def edge(eg, _):
                        for u in range(4):
                            ei = eg * 4 + u
                            we = wvs[b][pl.ds(ei, 16)][0]
                            for j in range(HCDIM // LANES):
                                v = rowss[b][ei, pl.ds(j * LANES, LANES)]
                                rowss[b][ei, pl.ds(j * LANES, LANES)] = (
                                    v * we)
                        return 0
                    lax.fori_loop(0, c2 // 4, edge, 0)

                    pltpu.async_copy(rowss[b], acc_sp.at[didxs[b]],
                                     ssems[b], add=True)
                return 0
            lax.fori_loop(0, nch // 2, pair, 0)
            pltpu.make_async_copy(rows0, acc_sp.at[didx0], ssem0).wait()
            pltpu.make_async_copy(rows1, acc_sp.at[didx1], ssem1).wait()
            plsc.subcore_barrier()

            for r0 in range(0, rows_per_sub, rb):
                row0 = pl.multiple_of(ss * rows_per_sub + r0, 8)
                row8 = pl.multiple_of((ss * rows_per_sub + r0) // 8, 8)
                pltpu.sync_copy(acc_sp.at[pl.ds(row0, rb)], eb_v)
                pltpu.sync_copy(den_hbm.at[0, pl.ds(row8, rb // 8)], d0_v)
                pltpu.sync_copy(den_hbm.at[1, pl.ds(row8, rb // 8)], d1_v)

                def erow(i, _):
                    pr = lax.shift_right_logical(i, 3)
                    pc = (i & 7) * 8
                    d0r = d0_v[pr, pl.ds(pc, 16)]
                    d1r = d1_v[pr, pl.ds(pc, 16)]
                    invv = 1.0 / (d0r + d1r + 1e-16)
                    inv = jnp.sum(jnp.where(IOTA() == head, invv, 0.0))
                    for j in range(HCDIM // LANES):
                        v = eb_v[i, pl.ds(j * LANES, LANES)]
                        eb_v[i, pl.ds(j * LANES, LANES)] = v * inv
                    return 0
                lax.fori_loop(0, rb, erow, 0)

                pltpu.sync_copy(
                    eb_v,
                    out_hbm.at[pl.ds(row0, rb),
                               pl.ds(pl.multiple_of(head * HCDIM, 128),
                                     HCDIM)])
            plsc.subcore_barrier()
            return 0
        lax.fori_loop(0, H // 2, headloop, 0)

    return k


def _make_pass3(n, n_pad):
    pslf = n_pad // NWK
    sbn = 80
    cn = 8
    prows = 128
    prow_per_sub = prows // NSUB

    @functools.partial(
        pl.kernel, mesh=_mesh(),
        compiler_params=pltpu.CompilerParams(needs_layout_passes=False),
        out_type=jax.ShapeDtypeStruct((2, prows, 128), F32),
        scratch_types=[
            pltpu.VMEM_SHARED((prows, 128), F32),
            pltpu.VMEM((cn, HHC), F32),
            pltpu.VMEM((sbn, 128), F32),
            pltpu.VMEM((sbn,), I32),
            pltpu.VMEM((HCDIM,), F32),
            pltpu.VMEM((HCDIM,), F32),
        ],
    )
    def k(agg_hbm, b_hbm, wl_hbm, batch_hbm, out_hbm,
          pool_sp, ab_v, pay_v, bidx, b4_v, wl_v):
        cc = lax.axis_index("c")
        ss = lax.axis_index("s")
        wid = _wid(cc, ss)
        nbase = wid * pslf
        zv16 = jnp.zeros((16,), F32)

        def zp(i, _):
            for j in range(8):
                pay_v[i, pl.ds(j * 16, 16)] = zv16
            return 0
        lax.fori_loop(0, sbn, zp, 0)

        pltpu.sync_copy(
            pay_v.at[pl.ds(0, prow_per_sub)],
            pool_sp.at[pl.ds(pl.multiple_of(ss * prow_per_sub, 8),
                             prow_per_sub)])
        pltpu.sync_copy(b_hbm, b4_v)
        pltpu.sync_copy(wl_hbm, wl_v)
        plsc.subcore_barrier()

        for sb in range(pslf // sbn):
            sb_base = nbase + sb * sbn
            pltpu.sync_copy(batch_hbm.at[pl.ds(sb_base, sbn)], bidx)

            def chunk(ci, _):
                pltpu.sync_copy(agg_hbm.at[pl.ds(sb_base + ci * cn, cn)],
                                ab_v)

                def node(i, _):
                    zacc = 0.0
                    for j in range(HCDIM // LANES):
                        acc = ab_v[i, pl.ds(j * LANES, LANES)]
                        for hh in range(1, H):
                            acc = acc + ab_v[i, pl.ds(hh * HCDIM + j * LANES,
                                                      LANES)]
                        o = acc * (1.0 / H) + b4_v[pl.ds(j * LANES, LANES)]
                        o = jnp.where(o > 0, o, jnp.exp(o) - 1.0)
                        zacc = zacc + jnp.sum(o * wl_v[pl.ds(j * LANES,
                                                             LANES)])
                    node_id = sb_base + ci * cn + i
                    validf = jnp.where(node_id < n, 1.0, 0.0)
                    lane = IOTA()
                    row = (jnp.where(lane == 0, zacc * validf, 0.0)
                           + jnp.where(lane == 1, validf, 0.0))
                    pay_v[ci * cn + i, pl.ds(0, 16)] = row.astype(F32)
                    return 0
                lax.fori_loop(0, cn, node, 0)
                return 0
            lax.fori_loop(0, sbn // cn, chunk, 0)

            pltpu.sync_copy(pay_v, pool_sp.at[bidx], add=True)
        plsc.subcore_barrier()
        prow = pl.multiple_of(ss * prow_per_sub, 8)
        pltpu.sync_copy(pool_sp.at[pl.ds(prow, prow_per_sub)],
                        out_hbm.at[cc, pl.ds(prow, prow_per_sub)])

    return k


def kernel(x, edge_index, edge_attr, batch,
           W1, We1, as1, ad1, ae1, b1,
           W2, We2, as2, ad2, ae2, b2,
           W3, We3, as3, ad3, ae3, b3,
           W4, We4, as4, ad4, ae4, b4,
           Wl, bl):
    n = x.shape[0]
    e = edge_index.shape[1]
    n_pad = ((n + NWK * LANES - 1) // (NWK * LANES)) * (NWK * LANES)
    e2p = e + n_pad
    assert e % (NWK * 400) == 0 and e2p % (NWK * 240) == 0

    src, dst = edge_index[0], edge_index[1]
    x_pad = jnp.pad(x, ((0, n_pad - n), (0, 0)))
    ii = jnp.arange(n_pad, dtype=I32)
    src2 = jnp.concatenate([src, jnp.where(ii < n, ii, 0)])
    dst2 = jnp.concatenate([dst, jnp.where(ii < n, ii, n_pad)])
    batch_pad = jnp.pad(batch, (0, n_pad - n), constant_values=NGRP)

    layers = [(W1, We1, as1, ad1, ae1, b1), (W2, We2, as2, ad2, ae2, b2),
              (W3, We3, as3, ad3, ae3, b3), (W4, We4, as4, ad4, ae4, b4)]

    ae_all = jnp.concatenate(
        [jnp.einsum('dhc,hc->dh', We.reshape(DE, H, HCDIM), ae)
         for (_, We, _, _, ae, _) in layers], axis=1)
    wcats = []
    for (W, _, a_s, a_d, _, _) in layers:
        k_in = W.shape[0]
        wa_s = jnp.einsum('ihc,hc->ih', W.reshape(k_in, H, HCDIM), a_s)
        wa_d = jnp.einsum('ihc,hc->ih', W.reshape(k_in, H, HCDIM), a_d)
        wcats.append(jnp.concatenate(
            [W, wa_s, wa_d, jnp.zeros((k_in, 1152 - HHC - 16), F32)], axis=1))

    qs = _ealpha_call(edge_attr, ae_all)
    p01 = _make_pass0(n, n_pad, e)(edge_attr.reshape(-1), dst)
    qls = _qloop_call(p01[0].reshape(n_pad, 32), p01[1].reshape(n_pad, 32),
                      ae_all, n_pad)

    pass1s = [_make_pass1(n, n_pad, e, e2p) for _ in range(4)]
    pass2 = _make_pass2(n_pad, e2p)

    agg = x_pad
    for li in range(4):
        bias_prev = (jnp.zeros((1, HHC), F32) if li == 0
                     else layers[li - 1][5].reshape(1, HHC))
        outs = _proj_call(agg, wcats[li], bias_prev, li > 0, n_pad)
        hs, st128, st16 = outs[:H], outs[H], outs[H + 1]
        w_flat, denp = pass1s[li](st128, st16.reshape(-1),
                                  qs[li].reshape(-1),
                                  qls[li].reshape(-1), src, dst)
        h3 = jnp.stack(hs)
        agg = pass2(h3, src2, dst2, w_flat, denp)

    pool = _make_pass3(n, n_pad)(agg, b4, Wl[:, 0], batch_pad)
    p = pool[0] + pool[1]
    return p[:NGRP, 0] / jnp.maximum(p[:NGRP, 1], 1.0) + bl[0]
